# Initial kernel scaffold; baseline (speedup 1.0000x reference)
import jax
import jax.numpy as jnp
from jax.experimental import pallas as pl


def kernel(inputs, embed0, embed1, idx):
    raise NotImplementedError("write your pallas kernel here")


# speedup vs baseline: 1.3247x; 1.3247x over previous
import functools

import jax
import jax.numpy as jnp
from jax import lax
from jax.experimental import pallas as pl
from jax.experimental.pallas import tpu as pltpu
from jax.experimental.pallas import tpu_sc as plsc

N_CODES_HALF = 1024
D = 64
K = 2 * N_CODES_HALF
T = 1024
GRID = 64
N_TOK = GRID * T
COMMITMENT_COST = 0.25

SC_CORES = 2
SC_SUBCORES = 16
NW = SC_CORES * SC_SUBCORES
B_PER_W = N_TOK // NW
CHUNK = 1024


def _tc_body(x_ref, codes_ref, pen_ref, idx_ref, loss_ref, perp_ref,
             hist_ref, acc_ref):
    b = pl.program_id(0)

    @pl.when(b == 0)
    def _init():
        hist_ref[...] = jnp.zeros_like(hist_ref)
        acc_ref[0, 0] = 0.0

    x = x_ref[0]
    codes = codes_ref[...]
    cnorm = jnp.sum(codes * codes, axis=1, keepdims=True)
    xc = jnp.dot(codes, x, preferred_element_type=jnp.float32)
    d = (cnorm + pen_ref[...]) - 2.0 * xc
    dmin = jnp.min(d, axis=0, keepdims=True)
    eq = d == dmin
    iota = lax.broadcasted_iota(jnp.int32, (K, T), 0)
    idx_ref[0, 0, :] = jnp.min(jnp.where(eq, iota, K), axis=0)
    hist_ref[...] += jnp.sum(eq.astype(jnp.float32), axis=1, keepdims=True)
    xnorm = jnp.sum(x * x, axis=0, keepdims=True)
    acc_ref[0, 0] += jnp.sum(dmin + xnorm)

    @pl.when(b == GRID - 1)
    def _fin():
        loss_ref[0, 0] = ((1.0 + COMMITMENT_COST) / (N_TOK * D)) * acc_ref[0, 0]
        p = hist_ref[...] * (1.0 / N_TOK)
        perp_ref[0, 0] = jnp.exp(-jnp.sum(p * jnp.log(p + 1e-10)))


@functools.lru_cache(maxsize=1)
def _sc_gather_fn():
    mesh = plsc.VectorSubcoreMesh(core_axis_name="c", subcore_axis_name="s")

    @functools.partial(
        pl.kernel,
        out_type=jax.ShapeDtypeStruct((N_TOK, D), jnp.float32),
        mesh=mesh,
        scratch_types=[
            pltpu.VMEM((CHUNK,), jnp.int32),
            pltpu.VMEM((CHUNK, D), jnp.float32),
            pltpu.SemaphoreType.DMA,
        ],
        compiler_params=pltpu.CompilerParams(use_tc_tiling_on_sc=False),
    )
    def _sc_gather(codes_hbm, idx_hbm, out_hbm, idx_v, rows_v, sem):
        wid = lax.axis_index("s") * SC_CORES + lax.axis_index("c")
        base = wid * B_PER_W
        for j in range(B_PER_W // CHUNK):
            off = base + j * CHUNK
            pltpu.sync_copy(idx_hbm.at[pl.ds(off, CHUNK)], idx_v)
            pltpu.async_copy(codes_hbm.at[idx_v], rows_v, sem).wait()
            pltpu.sync_copy(rows_v, out_hbm.at[pl.ds(off, CHUNK)])

    return _sc_gather


def kernel(inputs, embed0, embed1, idx):
    codes = jnp.concatenate([embed0, embed1], axis=0)
    half = (jnp.arange(K, dtype=jnp.int32) >= N_CODES_HALF)[:, None]
    pen = jnp.where(half & (idx == 0), jnp.inf, 0.0).astype(jnp.float32)
    x_r = inputs.reshape(GRID, D, T)

    indices, loss, perp = pl.pallas_call(
        _tc_body,
        grid=(GRID,),
        in_specs=[
            pl.BlockSpec((1, D, T), lambda b: (b, 0, 0)),
            pl.BlockSpec((K, D), lambda b: (0, 0)),
            pl.BlockSpec((K, 1), lambda b: (0, 0)),
        ],
        out_specs=[
            pl.BlockSpec((1, 1, T), lambda b: (b, 0, 0)),
            pl.BlockSpec(block_shape=(1, 1), index_map=lambda b: (0, 0),
                         memory_space=pltpu.SMEM),
            pl.BlockSpec(block_shape=(1, 1), index_map=lambda b: (0, 0),
                         memory_space=pltpu.SMEM),
        ],
        out_shape=[
            jax.ShapeDtypeStruct((GRID, 1, T), jnp.int32),
            jax.ShapeDtypeStruct((1, 1), jnp.float32),
            jax.ShapeDtypeStruct((1, 1), jnp.float32),
        ],
        scratch_shapes=[
            pltpu.VMEM((K, 1), jnp.float32),
            pltpu.SMEM((1, 1), jnp.float32),
        ],
    )(x_r, codes, pen)

    quantized = _sc_gather_fn()(codes, indices.reshape(N_TOK))
    q = quantized.reshape(GRID, 32, 32, D).transpose(0, 3, 1, 2)
    return (q, loss[0, 0], perp[0, 0])


# speedup vs baseline: 1.3915x; 1.0505x over previous
import functools

import jax
import jax.numpy as jnp
from jax import lax
from jax.experimental import pallas as pl
from jax.experimental.pallas import tpu as pltpu
from jax.experimental.pallas import tpu_sc as plsc

N_CODES_HALF = 1024
D = 64
K = 2 * N_CODES_HALF
T = 1024
GRID = 64
N_TOK = GRID * T
COMMITMENT_COST = 0.25

SC_CORES = 2
SC_SUBCORES = 16
NW = SC_CORES * SC_SUBCORES
B_PER_W = N_TOK // NW
CHUNK = 1024


def _tc_body(x_ref, codes_ref, bias_ref, idx_ref, hist_ref, acc_ref):
    b = pl.program_id(0)

    @pl.when(b == 0)
    def _init():
        hist_ref[...] = jnp.zeros_like(hist_ref)
        acc_ref[0, 0] = 0.0

    x = x_ref[0]
    xc = jnp.dot(codes_ref[...], x, preferred_element_type=jnp.float32)
    d = bias_ref[...] - 2.0 * xc
    dmin = jnp.min(d, axis=0, keepdims=True)
    iota = lax.broadcasted_iota(jnp.int32, (K, T), 0)
    idxi = jnp.min(jnp.where(d == dmin, iota, K), axis=0,
                   keepdims=True)
    idx_ref[0, 0, :] = idxi[0]
    onehot = (iota == idxi).astype(jnp.float32)
    hist_ref[...] += lax.dot_general(
        onehot, jnp.ones((T, 1), jnp.float32),
        (((1,), (0,)), ((), ())), preferred_element_type=jnp.float32)
    xnorm = jnp.sum(x * x, axis=0, keepdims=True)
    acc_ref[0, 0] += jnp.sum(dmin + xnorm)


def _fin_body(hist_ref, acc_ref, loss_ref, perp_ref):
    loss_ref[0, 0] = ((1.0 + COMMITMENT_COST) / (N_TOK * D)) * acc_ref[0, 0]
    p = hist_ref[...] * (1.0 / N_TOK)
    perp_ref[0, 0] = jnp.exp(-jnp.sum(p * jnp.log(p + 1e-10)))


@functools.lru_cache(maxsize=1)
def _sc_gather_fn():
    mesh = plsc.VectorSubcoreMesh(core_axis_name="c", subcore_axis_name="s")

    @functools.partial(
        pl.kernel,
        out_type=jax.ShapeDtypeStruct((N_TOK, D), jnp.float32),
        mesh=mesh,
        scratch_types=[
            pltpu.VMEM((CHUNK,), jnp.int32),
            pltpu.VMEM((CHUNK, D), jnp.float32),
            pltpu.SemaphoreType.DMA,
        ],
        compiler_params=pltpu.CompilerParams(use_tc_tiling_on_sc=False),
    )
    def _sc_gather(codes_hbm, idx_hbm, out_hbm, idx_v, rows_v, sem):
        wid = lax.axis_index("s") * SC_CORES + lax.axis_index("c")
        base = wid * B_PER_W
        for j in range(B_PER_W // CHUNK):
            off = base + j * CHUNK
            pltpu.sync_copy(idx_hbm.at[pl.ds(off, CHUNK)], idx_v)
            pltpu.async_copy(codes_hbm.at[idx_v], rows_v, sem).wait()
            pltpu.sync_copy(rows_v, out_hbm.at[pl.ds(off, CHUNK)])

    return _sc_gather


def kernel(inputs, embed0, embed1, idx):
    codes = jnp.concatenate([embed0, embed1], axis=0)
    cnorm = jnp.sum(codes * codes, axis=1, keepdims=True)
    half = (jnp.arange(K, dtype=jnp.int32) >= N_CODES_HALF)[:, None]
    pen = jnp.where(half & (idx == 0), jnp.inf, 0.0).astype(jnp.float32)
    bias = cnorm + pen
    x_r = inputs.reshape(GRID, D, T)

    indices, hist, acc = pl.pallas_call(
        _tc_body,
        grid=(GRID,),
        in_specs=[
            pl.BlockSpec((1, D, T), lambda b: (b, 0, 0)),
            pl.BlockSpec((K, D), lambda b: (0, 0)),
            pl.BlockSpec((K, 1), lambda b: (0, 0)),
        ],
        out_specs=[
            pl.BlockSpec((1, 1, T), lambda b: (b, 0, 0)),
            pl.BlockSpec((K, 1), lambda b: (0, 0)),
            pl.BlockSpec(block_shape=(1, 1), index_map=lambda b: (0, 0),
                         memory_space=pltpu.SMEM),
        ],
        out_shape=[
            jax.ShapeDtypeStruct((GRID, 1, T), jnp.int32),
            jax.ShapeDtypeStruct((K, 1), jnp.float32),
            jax.ShapeDtypeStruct((1, 1), jnp.float32),
        ],
    )(x_r, codes, bias)

    loss, perp = pl.pallas_call(
        _fin_body,
        in_specs=[
            pl.BlockSpec((K, 1), lambda: (0, 0)),
            pl.BlockSpec(block_shape=(1, 1), index_map=lambda: (0, 0),
                         memory_space=pltpu.SMEM),
        ],
        out_specs=[
            pl.BlockSpec(block_shape=(1, 1), index_map=lambda: (0, 0),
                         memory_space=pltpu.SMEM),
            pl.BlockSpec(block_shape=(1, 1), index_map=lambda: (0, 0),
                         memory_space=pltpu.SMEM),
        ],
        out_shape=[
            jax.ShapeDtypeStruct((1, 1), jnp.float32),
            jax.ShapeDtypeStruct((1, 1), jnp.float32),
        ],
    )(hist, acc)

    quantized = _sc_gather_fn()(codes, indices.reshape(N_TOK))
    q = quantized.reshape(GRID, 32, 32, D).transpose(0, 3, 1, 2)
    return (q, loss[0, 0], perp[0, 0])


# speedup vs baseline: 1.4692x; 1.0558x over previous
import functools

import jax
import jax.numpy as jnp
from jax import lax
from jax.experimental import pallas as pl
from jax.experimental.pallas import tpu as pltpu
from jax.experimental.pallas import tpu_sc as plsc

N_CODES_HALF = 1024
D = 64
K = 2 * N_CODES_HALF
T = 1024
GRID = 64
N_TOK = GRID * T
COMMITMENT_COST = 0.25

SC_CORES = 2
SC_SUBCORES = 16
NW = SC_CORES * SC_SUBCORES
B_PER_W = N_TOK // NW
CHUNK = 1024


def _tc_body(x_ref, codes2_ref, bias_ref, idx_ref, hist_ref, acc_ref):
    b = pl.program_id(0)

    @pl.when(b == 0)
    def _init():
        acc_ref[0, 0] = 0.0

    x = x_ref[0]
    xc2 = jnp.dot(codes2_ref[...], x, preferred_element_type=jnp.float32)
    d = bias_ref[...] - xc2
    dmin = jnp.min(d, axis=0, keepdims=True)
    eq = d == dmin
    iota = lax.broadcasted_iota(jnp.int32, (K, T), 0)
    idxi = jnp.min(jnp.where(eq, iota, K), axis=0,
                   keepdims=True)
    idx_ref[0, 0, :] = idxi[0]
    hist_ref[0, 0, :] = lax.dot_general(
        jnp.ones((1, T), jnp.float32), eq.astype(jnp.float32),
        (((1,), (1,)), ((), ())), preferred_element_type=jnp.float32)[0]
    xnorm = jnp.sum(x * x, axis=0, keepdims=True)
    acc_ref[0, 0] += jnp.sum(dmin + xnorm)


def _fin_body(hist_ref, acc_ref, loss_ref, perp_ref):
    loss_ref[0, 0] = ((1.0 + COMMITMENT_COST) / (N_TOK * D)) * acc_ref[0, 0]
    p = jnp.sum(hist_ref[:, 0, :], axis=0) * (1.0 / N_TOK)
    perp_ref[0, 0] = jnp.exp(-jnp.sum(p * jnp.log(p + 1e-10)))


@functools.lru_cache(maxsize=1)
def _sc_gather_fn():
    mesh = plsc.VectorSubcoreMesh(core_axis_name="c", subcore_axis_name="s")

    @functools.partial(
        pl.kernel,
        out_type=jax.ShapeDtypeStruct((N_TOK, D), jnp.float32),
        mesh=mesh,
        scratch_types=[
            pltpu.VMEM((CHUNK,), jnp.int32),
            pltpu.VMEM((CHUNK, D), jnp.float32),
            pltpu.SemaphoreType.DMA,
        ],
        compiler_params=pltpu.CompilerParams(use_tc_tiling_on_sc=False),
    )
    def _sc_gather(codes_hbm, idx_hbm, out_hbm, idx_v, rows_v, sem):
        wid = lax.axis_index("s") * SC_CORES + lax.axis_index("c")
        base = wid * B_PER_W
        for j in range(B_PER_W // CHUNK):
            off = base + j * CHUNK
            pltpu.sync_copy(idx_hbm.at[pl.ds(off, CHUNK)], idx_v)
            pltpu.async_copy(codes_hbm.at[idx_v], rows_v, sem).wait()
            pltpu.sync_copy(rows_v, out_hbm.at[pl.ds(off, CHUNK)])

    return _sc_gather


def kernel(inputs, embed0, embed1, idx):
    codes = jnp.concatenate([embed0, embed1], axis=0)
    cnorm = jnp.sum(codes * codes, axis=1, keepdims=True)
    half = (jnp.arange(K, dtype=jnp.int32) >= N_CODES_HALF)[:, None]
    pen = jnp.where(half & (idx == 0), jnp.inf, 0.0).astype(jnp.float32)
    bias = cnorm + pen
    x_r = inputs.reshape(GRID, D, T)

    indices, hist, acc = pl.pallas_call(
        _tc_body,
        grid=(GRID,),
        in_specs=[
            pl.BlockSpec((1, D, T), lambda b: (b, 0, 0)),
            pl.BlockSpec((K, D), lambda b: (0, 0)),
            pl.BlockSpec((K, 1), lambda b: (0, 0)),
        ],
        out_specs=[
            pl.BlockSpec((1, 1, T), lambda b: (b, 0, 0)),
            pl.BlockSpec((1, 1, K), lambda b: (b, 0, 0)),
            pl.BlockSpec(block_shape=(1, 1), index_map=lambda b: (0, 0),
                         memory_space=pltpu.SMEM),
        ],
        out_shape=[
            jax.ShapeDtypeStruct((GRID, 1, T), jnp.int32),
            jax.ShapeDtypeStruct((GRID, 1, K), jnp.float32),
            jax.ShapeDtypeStruct((1, 1), jnp.float32),
        ],
    )(x_r, 2.0 * codes, bias)

    loss, perp = pl.pallas_call(
        _fin_body,
        in_specs=[
            pl.BlockSpec((GRID, 1, K), lambda: (0, 0, 0)),
            pl.BlockSpec(block_shape=(1, 1), index_map=lambda: (0, 0),
                         memory_space=pltpu.SMEM),
        ],
        out_specs=[
            pl.BlockSpec(block_shape=(1, 1), index_map=lambda: (0, 0),
                         memory_space=pltpu.SMEM),
            pl.BlockSpec(block_shape=(1, 1), index_map=lambda: (0, 0),
                         memory_space=pltpu.SMEM),
        ],
        out_shape=[
            jax.ShapeDtypeStruct((1, 1), jnp.float32),
            jax.ShapeDtypeStruct((1, 1), jnp.float32),
        ],
    )(hist, acc)

    quantized = _sc_gather_fn()(codes, indices.reshape(N_TOK))
    q = quantized.reshape(GRID, 32, 32, D).transpose(0, 3, 1, 2)
    return (q, loss[0, 0], perp[0, 0])
